# baseline (speedup 1.0000x reference)
import functools

import jax
import jax.numpy as jnp
import numpy as np
from jax import lax
from jax.experimental import pallas as pl
from jax.experimental.pallas import tpu as pltpu
from jax.experimental.pallas import tpu_sc as plsc

_ROWS = 128
_N = 32768
_L = 16
_NV = _N // _L
_Q = 4
_NVQ = _NV // _Q
_SHIFTS = (0, 11, 22)
_MASKS = (0x7FF, 0x7FF, 0x3FF)
_RSIZE = (2048, 2048, 1024)
_NC = 2
_NS = 16
_ROWS_PER_W = _ROWS // (_NC * _NS)
_MININT = np.int32(-2147483648)


def _to_sortable(u):
    return u ^ (jnp.right_shift(u, 31) | _MININT)


def _from_sortable(u):
    return u ^ (jnp.right_shift(~u, 31) | _MININT)


def _digit(u, shift, mask):
    if shift:
        u = jnp.right_shift(u, np.int32(shift))
    return u & np.int32(mask)


def _last_lane(v):
    return lax.squeeze(lax.slice(v, (_L - 1,), (_L,)), (0,))


def _sort_body(x_hbm, out_hbm, a_v, b_v, h0, h1, h2, h3):
    wid = lax.axis_index("s") * _NC + lax.axis_index("c")
    hq = (h0, h1, h2, h3)
    zeros = jnp.zeros((_L,), jnp.int32)

    def do_row(r, _):
        row = wid * _ROWS_PER_W + r
        pltpu.sync_copy(x_hbm.at[row], a_v)

        for p in range(3):
            src, dst = (a_v, b_v) if p % 2 == 0 else (b_v, a_v)
            shift, mask, rsz = _SHIFTS[p], _MASKS[p], _RSIZE[p]

            def zero(j, _):
                sl = pl.ds(j * _L, _L)
                for q in range(_Q):
                    hq[q][sl] = zeros
                return 0

            lax.fori_loop(0, rsz // _L, zero, 0, unroll=8)

            def hist(i, _, src=src, shift=shift, mask=mask, p=p):
                for q in range(_Q):
                    sl = pl.ds((q * _NVQ + i) * _L, _L)
                    u = plsc.bitcast(src[sl], jnp.int32)
                    if p == 0:
                        u = _to_sortable(u)
                        src[sl] = plsc.bitcast(u, jnp.float32)
                    d = _digit(u, shift, mask)
                    cnt, last = plsc.scan_count(d)
                    plsc.addupdate_scatter(hq[q], [d], cnt, mask=last)
                return 0

            lax.fori_loop(0, _NVQ, hist, 0)

            def prefix(j, carry):
                sl = pl.ds(j * _L, _L)
                v = [h[sl] for h in hq]
                t = (v[0] + v[1]) + (v[2] + v[3])
                c = plsc.cumsum(t)
                excl = c - t + carry
                for q in range(_Q):
                    hq[q][sl] = excl
                    if q < _Q - 1:
                        excl = excl + v[q]
                return carry + _last_lane(c)

            lax.fori_loop(0, rsz // _L, prefix, jnp.int32(0), unroll=2)

            def permute(i, _, src=src, dst=dst, shift=shift, mask=mask,
                        p=p):
                for q in range(_Q):
                    u = plsc.bitcast(src[pl.ds((q * _NVQ + i) * _L, _L)],
                                     jnp.int32)
                    d = _digit(u, shift, mask)
                    cnt, last = plsc.scan_count(d)
                    base = plsc.load_gather(hq[q], [d])
                    nxt = base + cnt
                    pos = nxt - 1
                    out = _from_sortable(u) if p == 2 else u
                    plsc.store_scatter(dst, [pos],
                                       plsc.bitcast(out, jnp.float32))
                    plsc.store_scatter(hq[q], [d], nxt, mask=last)
                return 0

            lax.fori_loop(0, _NVQ, permute, 0, unroll=2)

        pltpu.sync_copy(b_v, out_hbm.at[row])
        return 0

    lax.fori_loop(0, _ROWS_PER_W, do_row, 0)


@jax.jit
def kernel(x):
    mesh = plsc.VectorSubcoreMesh(
        core_axis_name="c", subcore_axis_name="s", num_cores=_NC,
        num_subcores=_NS)
    run = pl.kernel(
        _sort_body,
        out_type=jax.ShapeDtypeStruct((_ROWS, _N), jnp.float32),
        mesh=mesh,
        scratch_types=[
            pltpu.VMEM((_N,), jnp.float32),
            pltpu.VMEM((_N,), jnp.float32),
        ] + [pltpu.VMEM((2048,), jnp.int32)] * 4,
        compiler_params=pltpu.CompilerParams(needs_layout_passes=False),
    )
    return run(x)


# speedup vs baseline: 1.6009x; 1.6009x over previous
import functools

import jax
import jax.numpy as jnp
import numpy as np
from jax import lax
from jax.experimental import pallas as pl
from jax.experimental.pallas import tpu as pltpu
from jax.experimental.pallas import tpu_sc as plsc

_ROWS = 128
_N = 32768
_L = 16
_NV = _N // _L
_SHIFTS = (0, 11, 22)
_MASKS = (0x7FF, 0x7FF, 0x3FF)
_RSIZE = (2048, 2048, 1024)
_NC = 2
_NS = 16
_ROWS_PER_W = _ROWS // (_NC * _NS)
_MININT = np.int32(-2147483648)


def _to_sortable(u):
    return u ^ (jnp.right_shift(u, 31) | _MININT)


def _from_sortable(u):
    return u ^ (jnp.right_shift(~u, 31) | _MININT)


def _digit(u, shift, mask):
    if shift:
        u = jnp.right_shift(u, np.int32(shift))
    return u & np.int32(mask)


def _last_lane(v):
    return lax.squeeze(lax.slice(v, (_L - 1,), (_L,)), (0,))


def _sort_body(x_hbm, out_hbm, a_v, b_v, h0_v, h1_v, h2_v):
    wid = lax.axis_index("s") * _NC + lax.axis_index("c")
    hists = (h0_v, h1_v, h2_v)
    zeros = jnp.zeros((_L,), jnp.int32)
    ones = jnp.ones((_L,), jnp.int32)

    def do_row(r, _):
        row = wid * _ROWS_PER_W + r
        pltpu.sync_copy(x_hbm.at[row], a_v)

        def zero_all(j, _):
            sl = pl.ds(j * _L, _L)
            h0_v[sl] = zeros
            h1_v[sl] = zeros
            return 0

        def zero_all2(j, _):
            sl = pl.ds(j * _L, _L)
            h0_v[sl] = zeros
            h1_v[sl] = zeros
            h2_v[sl] = zeros
            return 0

        lax.fori_loop(0, 1024 // _L, zero_all2, 0, unroll=8)
        lax.fori_loop(1024 // _L, 2048 // _L, zero_all, 0, unroll=8)

        def hist_all(i, _):
            sl = pl.ds(i * _L, _L)
            u = plsc.bitcast(a_v[sl], jnp.int32)
            u = _to_sortable(u)
            a_v[sl] = plsc.bitcast(u, jnp.float32)
            for p in range(3):
                d = _digit(u, _SHIFTS[p], _MASKS[p])
                plsc.addupdate_scatter(hists[p], [d], ones)
            return 0

        lax.fori_loop(0, _NV, hist_all, 0, unroll=2)

        for p in range(3):
            src, dst = (a_v, b_v) if p % 2 == 0 else (b_v, a_v)
            shift, mask, hist = _SHIFTS[p], _MASKS[p], hists[p]

            def prefix(j, carry, hist=hist):
                sl = pl.ds(j * _L, _L)
                v = hist[sl]
                c = plsc.cumsum(v)
                hist[sl] = c - v + carry
                return carry + _last_lane(c)

            lax.fori_loop(0, _RSIZE[p] // _L, prefix, jnp.int32(0),
                          unroll=2)

            def permute(i, _, src=src, dst=dst, shift=shift, mask=mask,
                        hist=hist, p=p):
                u = plsc.bitcast(src[pl.ds(i * _L, _L)], jnp.int32)
                d = _digit(u, shift, mask)
                cnt, last = plsc.scan_count(d)
                base = plsc.load_gather(hist, [d])
                nxt = base + cnt
                pos = nxt - 1
                out = _from_sortable(u) if p == 2 else u
                plsc.store_scatter(dst, [pos], plsc.bitcast(out, jnp.float32))
                plsc.store_scatter(hist, [d], nxt, mask=last)
                return 0

            lax.fori_loop(0, _NV, permute, 0, unroll=4)

        pltpu.sync_copy(b_v, out_hbm.at[row])
        return 0

    lax.fori_loop(0, _ROWS_PER_W, do_row, 0)


@jax.jit
def kernel(x):
    mesh = plsc.VectorSubcoreMesh(
        core_axis_name="c", subcore_axis_name="s", num_cores=_NC,
        num_subcores=_NS)
    run = pl.kernel(
        _sort_body,
        out_type=jax.ShapeDtypeStruct((_ROWS, _N), jnp.float32),
        mesh=mesh,
        scratch_types=[
            pltpu.VMEM((_N,), jnp.float32),
            pltpu.VMEM((_N,), jnp.float32),
            pltpu.VMEM((2048,), jnp.int32),
            pltpu.VMEM((2048,), jnp.int32),
            pltpu.VMEM((1024,), jnp.int32),
        ],
        compiler_params=pltpu.CompilerParams(needs_layout_passes=False),
    )
    return run(x)


# speedup vs baseline: 3.1894x; 1.9922x over previous
import functools

import jax
import jax.numpy as jnp
import numpy as np
from jax import lax
from jax.experimental import pallas as pl
from jax.experimental.pallas import tpu as pltpu
from jax.experimental.pallas import tpu_sc as plsc

_ROWS = 128
_N = 32768
_L = 16
_NV = _N // _L
_Q = 4
_NVQ = _NV // _Q
_QSHIFT = 13
_SHIFTS = (0, 11, 22)
_MASKS = (0x7FF, 0x7FF, 0x3FF)
_RSIZE = (2048, 2048, 1024)
_RBITS = (11, 11, 10)
_NC = 2
_NS = 16
_ROWS_PER_W = _ROWS // (_NC * _NS)
_MININT = np.int32(-2147483648)


def _to_sortable(u):
    return u ^ (jnp.right_shift(u, 31) | _MININT)


def _from_sortable(u):
    return u ^ (jnp.right_shift(~u, 31) | _MININT)


def _digit(u, shift, mask):
    if shift:
        u = jnp.right_shift(u, np.int32(shift))
    return u & np.int32(mask)


def _last_lane(v):
    return lax.squeeze(lax.slice(v, (_L - 1,), (_L,)), (0,))


def _sort_body(x_hbm, out_hbm, a_v, b_v, o0, o1, o2, o3, nh_v):
    wid = lax.axis_index("s") * _NC + lax.axis_index("c")
    offs = (o0, o1, o2, o3)
    zeros = jnp.zeros((_L,), jnp.int32)
    ones = jnp.ones((_L,), jnp.int32)

    def do_row(r, _):
        row = wid * _ROWS_PER_W + r
        pltpu.sync_copy(x_hbm.at[row], a_v)

        def zero_nh(j, _):
            nh_v[pl.ds(j * _L, _L)] = zeros
            return 0

        lax.fori_loop(0, _Q * 2048 // _L, zero_nh, 0, unroll=8)

        def hist0(i, _):
            for q in range(_Q):
                sl = pl.ds((q * _NVQ + i) * _L, _L)
                u = plsc.bitcast(a_v[sl], jnp.int32)
                u = _to_sortable(u)
                a_v[sl] = plsc.bitcast(u, jnp.float32)
                d = _digit(u, _SHIFTS[0], _MASKS[0])
                plsc.addupdate_scatter(nh_v, [d + np.int32(q * _RSIZE[0])],
                                       ones)
            return 0

        lax.fori_loop(0, _NVQ, hist0, 0)

        for p in range(3):
            src, dst = (a_v, b_v) if p % 2 == 0 else (b_v, a_v)
            shift, mask, rsz = _SHIFTS[p], _MASKS[p], _RSIZE[p]

            def prefix(j, carry, rsz=rsz):
                sl = [pl.ds(q * rsz + j * _L, _L) for q in range(_Q)]
                v = [nh_v[s] for s in sl]
                t = (v[0] + v[1]) + (v[2] + v[3])
                c = plsc.cumsum(t)
                excl = c - t + carry
                jl = pl.ds(j * _L, _L)
                for q in range(_Q):
                    offs[q][jl] = excl
                    nh_v[sl[q]] = zeros
                    if q < _Q - 1:
                        excl = excl + v[q]
                return carry + _last_lane(c)

            lax.fori_loop(0, rsz // _L, prefix, jnp.int32(0), unroll=2)

            def permute(i, _, src=src, dst=dst, shift=shift, mask=mask,
                        p=p):
                us = [plsc.bitcast(src[pl.ds((q * _NVQ + i) * _L, _L)],
                                   jnp.int32) for q in range(_Q)]
                ds_ = [_digit(u, shift, mask) for u in us]
                scans = [plsc.scan_count(d) for d in ds_]
                bases = [plsc.load_gather(offs[q], [ds_[q]])
                         for q in range(_Q)]
                nxts = [bases[q] + scans[q][0] for q in range(_Q)]
                poss = [nxt - 1 for nxt in nxts]
                outs = [_from_sortable(u) if p == 2 else u for u in us]
                for q in range(_Q):
                    plsc.store_scatter(dst, [poss[q]],
                                       plsc.bitcast(outs[q], jnp.float32))
                    plsc.store_scatter(offs[q], [ds_[q]], nxts[q],
                                       mask=scans[q][1])
                if p < 2:
                    for q in range(_Q):
                        dn = _digit(us[q], _SHIFTS[p + 1], _MASKS[p + 1])
                        qn = jnp.left_shift(
                            jnp.right_shift(poss[q], np.int32(_QSHIFT)),
                            np.int32(_RBITS[p + 1]))
                        plsc.addupdate_scatter(nh_v, [qn + dn], ones)
                return 0

            lax.fori_loop(0, _NVQ, permute, 0)

        pltpu.sync_copy(b_v, out_hbm.at[row])
        return 0

    lax.fori_loop(0, _ROWS_PER_W, do_row, 0)


@jax.jit
def kernel(x):
    mesh = plsc.VectorSubcoreMesh(
        core_axis_name="c", subcore_axis_name="s", num_cores=_NC,
        num_subcores=_NS)
    run = pl.kernel(
        _sort_body,
        out_type=jax.ShapeDtypeStruct((_ROWS, _N), jnp.float32),
        mesh=mesh,
        scratch_types=[
            pltpu.VMEM((_N,), jnp.float32),
            pltpu.VMEM((_N,), jnp.float32),
        ] + [pltpu.VMEM((2048,), jnp.int32)] * 4
          + [pltpu.VMEM((_Q * 2048,), jnp.int32)],
        compiler_params=pltpu.CompilerParams(needs_layout_passes=False),
    )
    return run(x)
